# baseline (speedup 1.0000x reference)
import numpy as np
import jax
import jax.numpy as jnp
from jax.experimental import pallas as pl
from jax.experimental.pallas import tpu as pltpu

_N = 4096
_NFEAT = 128
_NHID = 64
_HID2 = 32
_BLK = 512
_G = _N // _BLK


def _build_schedule():
    rows = []
    for s in range(_G):
        rows.append([s, 0, 0, 0, 0])
    for j in range(_G):
        rows.append([_G - 1, j, 0, 1, j])
        for t in range(1, j + 1):
            rows.append([_G - 1, j, t, 0, j])
        for i in range(j):
            rows.append([_G - 1, i, j, 0, j])
    return np.asarray(rows, dtype=np.int32)


_SCHED = _build_schedule()
_STEPS = _SCHED.shape[0]


def _gcn_kernel(t_ref, x_ref, adj_ref, w1_ref, b1_ref, w2_ref, b2_ref,
                z_ref, recon_ref,
                adj_scr, s1_scr, h_scr, s2_scr, zbf_scr, zt_scr):
    s = pl.program_id(0)
    a = t_ref[s, 1]
    b = t_ref[s, 2]
    isdiag = t_ref[s, 3]

    @pl.when(s == 0)
    def _():
        s1 = jnp.dot(x_ref[...], w1_ref[...],
                     preferred_element_type=jnp.float32)
        s1_scr[...] = s1.astype(jnp.bfloat16)

    @pl.when(s < _G)
    def _():
        blk = adj_ref[...].astype(jnp.bfloat16)
        adj_scr[pl.ds(s * _BLK, _BLK), :] = blk
        h = jnp.dot(blk, s1_scr[...],
                    preferred_element_type=jnp.float32) + b1_ref[...]
        h_scr[pl.ds(s * _BLK, _BLK), :] = jnp.maximum(h, 0.0).astype(jnp.bfloat16)

    @pl.when(s == _G - 1)
    def _():
        s2 = jnp.dot(h_scr[...], w2_ref[...].astype(jnp.bfloat16),
                     preferred_element_type=jnp.float32)
        s2_scr[...] = s2.astype(jnp.bfloat16)

    @pl.when(jnp.logical_and(s >= _G, isdiag == 1))
    def _():
        zj = jnp.dot(adj_scr[pl.ds(a * _BLK, _BLK), :], s2_scr[...],
                     preferred_element_type=jnp.float32) + b2_ref[...]
        z_ref[...] = zj
        zj_bf = zj.astype(jnp.bfloat16)
        zbf_scr[pl.ds(a * _BLK, _BLK), :] = zj_bf
        zt_scr[:, pl.ds(a * _BLK, _BLK)] = zj_bf.T

    @pl.when(s >= _G)
    def _():
        recon_ref[...] = jnp.dot(zbf_scr[pl.ds(a * _BLK, _BLK), :],
                                 zt_scr[:, pl.ds(b * _BLK, _BLK)],
                                 preferred_element_type=jnp.float32)


def kernel(x, adj, W1, b1, W2, b2):
    b1r = b1.reshape(1, _NHID)
    b2r = b2.reshape(1, _HID2)
    sched = jnp.asarray(_SCHED)

    grid_spec = pltpu.PrefetchScalarGridSpec(
        num_scalar_prefetch=1,
        grid=(_STEPS,),
        in_specs=[
            pl.BlockSpec((_N, _NFEAT), lambda s, t: (0, 0)),
            pl.BlockSpec((_BLK, _N), lambda s, t: (t[s, 0], 0)),
            pl.BlockSpec((_NFEAT, _NHID), lambda s, t: (0, 0)),
            pl.BlockSpec((1, _NHID), lambda s, t: (0, 0)),
            pl.BlockSpec((_NHID, _HID2), lambda s, t: (0, 0)),
            pl.BlockSpec((1, _HID2), lambda s, t: (0, 0)),
        ],
        out_specs=[
            pl.BlockSpec((_BLK, _HID2), lambda s, t: (t[s, 4], 0)),
            pl.BlockSpec((_BLK, _BLK), lambda s, t: (t[s, 1], t[s, 2])),
        ],
        scratch_shapes=[
            pltpu.VMEM((_N, _N), jnp.bfloat16),
            pltpu.VMEM((_N, _NHID), jnp.bfloat16),
            pltpu.VMEM((_N, _NHID), jnp.bfloat16),
            pltpu.VMEM((_N, _HID2), jnp.bfloat16),
            pltpu.VMEM((_N, _HID2), jnp.bfloat16),
            pltpu.VMEM((_HID2, _N), jnp.bfloat16),
        ],
    )

    z, recon = pl.pallas_call(
        _gcn_kernel,
        grid_spec=grid_spec,
        out_shape=[
            jax.ShapeDtypeStruct((_N, _HID2), jnp.float32),
            jax.ShapeDtypeStruct((_N, _N), jnp.float32),
        ],
        compiler_params=pltpu.CompilerParams(
            dimension_semantics=("arbitrary",)),
    )(sched, x, adj, W1, b1r, W2, b2r)

    return (recon, z)


# speedup vs baseline: 1.0361x; 1.0361x over previous
import numpy as np
import jax
import jax.numpy as jnp
from jax.experimental import pallas as pl
from jax.experimental.pallas import tpu as pltpu

_N = 4096
_NFEAT = 128
_NHID = 64
_HID2 = 32
_BLK = 256
_G = _N // _BLK
_HALF = _N // 2
_GH = _G // 2


def _build_schedule():
    rows = []
    for s in range(_G):
        rows.append([s, 0, 0, 0, 0, 0])
    for j in range(_GH):
        rows.append([_G - 1, 0, 0, 0, j, 1])
    for m in range(_GH):
        rows.append([_G - 1, m, 0, 1, _GH - 1 + m, 0])
        rows.append([_G - 1, m, 0, 0, _GH + m, 1])
    for m in range(_GH, _G):
        rows.append([_G - 1, m, 0, 1, _G - 1, 0])
    for m in range(_G):
        rows.append([_G - 1, m, 1, 1, _G - 1, 0])
    return np.asarray(rows, dtype=np.int32)


_SCHED = _build_schedule()
_STEPS = _SCHED.shape[0]


def _gcn_kernel(t_ref, x_ref, adj_ref, w1_ref, b1_ref, w2_ref, b2_ref,
                z_ref, recon_ref,
                adj_scr, s1_scr, h_scr, s2_scr, zbf_scr, zt_scr):
    s = pl.program_id(0)
    ar = t_ref[s, 1]
    hr = t_ref[s, 2]
    do_recon = t_ref[s, 3]
    jz = t_ref[s, 4]
    do_z = t_ref[s, 5]

    @pl.when(s == 0)
    def _():
        s1 = jnp.dot(x_ref[...], w1_ref[...],
                     preferred_element_type=jnp.float32)
        s1_scr[...] = s1.astype(jnp.bfloat16)

    @pl.when(s < _G)
    def _():
        blk = adj_ref[...].astype(jnp.bfloat16)
        adj_scr[pl.ds(s * _BLK, _BLK), :] = blk
        h = jnp.dot(blk, s1_scr[...],
                    preferred_element_type=jnp.float32) + b1_ref[...]
        h_scr[pl.ds(s * _BLK, _BLK), :] = jnp.maximum(h, 0.0).astype(jnp.bfloat16)

    @pl.when(s == _G - 1)
    def _():
        s2 = jnp.dot(h_scr[...], w2_ref[...].astype(jnp.bfloat16),
                     preferred_element_type=jnp.float32)
        s2_scr[...] = s2.astype(jnp.bfloat16)

    @pl.when(do_z == 1)
    def _():
        zj = jnp.dot(adj_scr[pl.ds(jz * _BLK, _BLK), :], s2_scr[...],
                     preferred_element_type=jnp.float32) + b2_ref[...]
        z_ref[...] = zj
        zj_bf = zj.astype(jnp.bfloat16)
        zbf_scr[pl.ds(jz * _BLK, _BLK), :] = zj_bf
        zt_scr[:, pl.ds(jz * _BLK, _BLK)] = zj_bf.T

    @pl.when(do_recon == 1)
    def _():
        recon_ref[...] = jnp.dot(zbf_scr[pl.ds(ar * _BLK, _BLK), :],
                                 zt_scr[:, pl.ds(hr * _HALF, _HALF)],
                                 preferred_element_type=jnp.float32)


def kernel(x, adj, W1, b1, W2, b2):
    b1r = b1.reshape(1, _NHID)
    b2r = b2.reshape(1, _HID2)
    sched = jnp.asarray(_SCHED)

    grid_spec = pltpu.PrefetchScalarGridSpec(
        num_scalar_prefetch=1,
        grid=(_STEPS,),
        in_specs=[
            pl.BlockSpec((_N, _NFEAT), lambda s, t: (0, 0)),
            pl.BlockSpec((_BLK, _N), lambda s, t: (t[s, 0], 0)),
            pl.BlockSpec((_NFEAT, _NHID), lambda s, t: (0, 0)),
            pl.BlockSpec((1, _NHID), lambda s, t: (0, 0)),
            pl.BlockSpec((_NHID, _HID2), lambda s, t: (0, 0)),
            pl.BlockSpec((1, _HID2), lambda s, t: (0, 0)),
        ],
        out_specs=[
            pl.BlockSpec((_BLK, _HID2), lambda s, t: (t[s, 4], 0)),
            pl.BlockSpec((_BLK, _HALF), lambda s, t: (t[s, 1], t[s, 2])),
        ],
        scratch_shapes=[
            pltpu.VMEM((_N, _N), jnp.bfloat16),
            pltpu.VMEM((_N, _NHID), jnp.bfloat16),
            pltpu.VMEM((_N, _NHID), jnp.bfloat16),
            pltpu.VMEM((_N, _HID2), jnp.bfloat16),
            pltpu.VMEM((_N, _HID2), jnp.bfloat16),
            pltpu.VMEM((_HID2, _N), jnp.bfloat16),
        ],
    )

    z, recon = pl.pallas_call(
        _gcn_kernel,
        grid_spec=grid_spec,
        out_shape=[
            jax.ShapeDtypeStruct((_N, _HID2), jnp.float32),
            jax.ShapeDtypeStruct((_N, _N), jnp.float32),
        ],
        compiler_params=pltpu.CompilerParams(
            dimension_semantics=("arbitrary",)),
    )(sched, x, adj, W1, b1r, W2, b2r)

    return (recon, z)


# speedup vs baseline: 1.2591x; 1.2152x over previous
import jax
import jax.numpy as jnp
from jax.experimental import pallas as pl
from jax.experimental.pallas import tpu as pltpu

_N = 4096
_NFEAT = 128
_NHID = 64
_HID2 = 32

_B1 = 256
_G1 = _N // _B1
_B2 = 1024
_G2 = _N // _B2
_B3 = 256
_G3 = _N // _B3
_STEPS = _G1 + _G2 + _G3


def _gcn_kernel(x_ref, adj_ref, w1_ref, b1_ref, w2_ref, b2_ref,
                z_ref, recon_ref,
                adj_scr, s1_scr, h_scr, s2_scr, zbf_scr, zt_scr):
    i = pl.program_id(0)

    @pl.when(i == 0)
    def _():
        s1 = jnp.dot(x_ref[...], w1_ref[...],
                     preferred_element_type=jnp.float32)
        s1_scr[...] = s1.astype(jnp.bfloat16)

    @pl.when(i < _G1)
    def _():
        blk = adj_ref[...].astype(jnp.bfloat16)
        adj_scr[pl.ds(i * _B1, _B1), :] = blk
        h = jnp.dot(blk, s1_scr[...],
                    preferred_element_type=jnp.float32) + b1_ref[...]
        h_scr[pl.ds(i * _B1, _B1), :] = jnp.maximum(h, 0.0).astype(jnp.bfloat16)

    @pl.when(i == _G1 - 1)
    def _():
        s2 = jnp.dot(h_scr[...], w2_ref[...].astype(jnp.bfloat16),
                     preferred_element_type=jnp.float32)
        s2_scr[...] = s2.astype(jnp.bfloat16)

    @pl.when(jnp.logical_and(i >= _G1, i < _G1 + _G2))
    def _():
        j = i - _G1
        zj = jnp.dot(adj_scr[pl.ds(j * _B2, _B2), :], s2_scr[...],
                     preferred_element_type=jnp.float32) + b2_ref[...]
        z_ref[...] = zj
        zj_bf = zj.astype(jnp.bfloat16)
        zbf_scr[pl.ds(j * _B2, _B2), :] = zj_bf
        zt_scr[:, pl.ds(j * _B2, _B2)] = zj_bf.T

    @pl.when(i >= _G1 + _G2)
    def _():
        k = i - _G1 - _G2
        recon_ref[...] = jnp.dot(zbf_scr[pl.ds(k * _B3, _B3), :],
                                 zt_scr[...],
                                 preferred_element_type=jnp.float32)


def kernel(x, adj, W1, b1, W2, b2):
    b1r = b1.reshape(1, _NHID)
    b2r = b2.reshape(1, _HID2)

    z, recon = pl.pallas_call(
        _gcn_kernel,
        grid=(_STEPS,),
        in_specs=[
            pl.BlockSpec((_N, _NFEAT), lambda i: (0, 0)),
            pl.BlockSpec((_B1, _N), lambda i: (jnp.minimum(i, _G1 - 1), 0)),
            pl.BlockSpec((_NFEAT, _NHID), lambda i: (0, 0)),
            pl.BlockSpec((1, _NHID), lambda i: (0, 0)),
            pl.BlockSpec((_NHID, _HID2), lambda i: (0, 0)),
            pl.BlockSpec((1, _HID2), lambda i: (0, 0)),
        ],
        out_specs=[
            pl.BlockSpec((_B2, _HID2),
                         lambda i: (jnp.clip(i - _G1, 0, _G2 - 1), 0)),
            pl.BlockSpec((_B3, _N),
                         lambda i: (jnp.clip(i - _G1 - _G2, 0, _G3 - 1), 0)),
        ],
        out_shape=[
            jax.ShapeDtypeStruct((_N, _HID2), jnp.float32),
            jax.ShapeDtypeStruct((_N, _N), jnp.float32),
        ],
        scratch_shapes=[
            pltpu.VMEM((_N, _N), jnp.bfloat16),
            pltpu.VMEM((_N, _NHID), jnp.bfloat16),
            pltpu.VMEM((_N, _NHID), jnp.bfloat16),
            pltpu.VMEM((_N, _HID2), jnp.bfloat16),
            pltpu.VMEM((_N, _HID2), jnp.bfloat16),
            pltpu.VMEM((_HID2, _N), jnp.bfloat16),
        ],
        compiler_params=pltpu.CompilerParams(
            dimension_semantics=("arbitrary",)),
    )(x, adj, W1, b1r, W2, b2r)

    return (recon, z)
